# baseline (speedup 1.0000x reference)
import functools

import jax
import jax.numpy as jnp
from jax import lax
from jax.experimental import pallas as pl
from jax.experimental.pallas import tpu as pltpu
from jax.experimental.pallas import tpu_sc as plsc


@functools.lru_cache(maxsize=None)
def _build(B, C, D, R):
    info = plsc.get_sparse_core_info()
    nw = info.num_cores * info.num_subcores
    while B % nw != 0:
        nw //= 2
    bpw = B // nw
    nc = info.num_cores
    nslots = 2 if bpw > 1 else 1

    mesh = plsc.VectorSubcoreMesh(core_axis_name="c", subcore_axis_name="s")

    @functools.partial(
        pl.kernel,
        mesh=mesh,
        compiler_params=pltpu.CompilerParams(use_tc_tiling_on_sc=False),
        out_type=jax.ShapeDtypeStruct((B, R + C, D), jnp.float32),
        scratch_types=[
            pltpu.VMEM((R,), jnp.int32),
            pltpu.VMEM((nslots, R + C, D), jnp.float32),
            pltpu.SemaphoreType.DMA,
            pltpu.SemaphoreType.DMA,
            pltpu.SemaphoreType.DMA,
        ],
    )
    def sc_concat(x_hbm, idx_hbm, emb_hbm, out_hbm, idx_v, slots_v, gsem, s0, s1):
        wid = lax.axis_index("s") * nc + lax.axis_index("c")
        ssem = [s0, s1]

        @pl.when(wid < nw)
        def _():
            base = wid * bpw
            pltpu.sync_copy(idx_hbm, idx_v)
            for j in range(nslots):
                pltpu.async_copy(
                    emb_hbm.at[idx_v], slots_v.at[j, pl.ds(0, R)], gsem
                ).wait()

            def start_in(i):
                return pltpu.async_copy(
                    x_hbm.at[base + i],
                    slots_v.at[i % nslots, pl.ds(R, C)],
                    ssem[i % nslots],
                )

            def start_out(i):
                return pltpu.async_copy(
                    slots_v.at[i % nslots], out_hbm.at[base + i], ssem[i % nslots]
                )

            h_in = [None] * bpw
            h_out = [None] * bpw
            for i in range(bpw):
                if i >= nslots:
                    h_out[i - nslots].wait()
                h_in[i] = start_in(i)
                if i >= 1:
                    h_in[i - 1].wait()
                    h_out[i - 1] = start_out(i - 1)
            h_in[bpw - 1].wait()
            h_out[bpw - 1] = start_out(bpw - 1)
            for i in range(max(0, bpw - nslots), bpw):
                h_out[i].wait()

    return sc_concat


def kernel(x, y, emb_table):
    B, C = x.shape[0], x.shape[1]
    R, D = emb_table.shape
    x3 = x.reshape(B, C, D)
    idx = y.reshape(-1).astype(jnp.int32)
    return _build(B, C, D, int(idx.shape[0]))(x3, idx, emb_table)


# speedup vs baseline: 1.2651x; 1.2651x over previous
import functools

import jax
import jax.numpy as jnp
from jax import lax
from jax.experimental import pallas as pl
from jax.experimental.pallas import tpu as pltpu
from jax.experimental.pallas import tpu_sc as plsc

_TILE = 8


@functools.lru_cache(maxsize=None)
def _build(B, C, D, R):
    info = plsc.get_sparse_core_info()
    nw = info.num_cores * info.num_subcores
    while B % nw != 0:
        nw //= 2
    bpw = B // nw
    nc = info.num_cores
    nlanes = info.num_lanes

    mesh = plsc.VectorSubcoreMesh(core_axis_name="c", subcore_axis_name="s")

    @functools.partial(
        pl.kernel,
        mesh=mesh,
        out_type=jax.ShapeDtypeStruct((B, R + C, D), jnp.float32),
        scratch_types=[
            pltpu.VMEM((_TILE,), jnp.int32),
            pltpu.VMEM((C, D), jnp.float32),
            pltpu.VMEM((R + C, D), jnp.float32),
            pltpu.SemaphoreType.DMA,
            pltpu.SemaphoreType.DMA,
            pltpu.SemaphoreType.DMA,
        ],
    )
    def sc_concat(x_hbm, idx_hbm, emb_hbm, out_hbm, idx_v, x_v, o_v, gsem, isem, osem):
        wid = lax.axis_index("s") * nc + lax.axis_index("c")

        @pl.when(wid < nw)
        def _():
            base = wid * bpw
            pltpu.sync_copy(idx_hbm, idx_v)
            pltpu.async_copy(emb_hbm.at[idx_v], o_v.at[pl.ds(0, _TILE)], gsem).wait()

            def start_in(i):
                return pltpu.async_copy(x_hbm.at[base + i], x_v, isem)

            def start_out(i):
                return pltpu.async_copy(o_v, out_hbm.at[base + i], osem)

            def shift():
                @plsc.parallel_loop(0, C)
                def _row(r):
                    for c in range(D // nlanes):
                        o_v[R + r, pl.ds(c * nlanes, nlanes)] = x_v[
                            r, pl.ds(c * nlanes, nlanes)
                        ]

            h_in = start_in(0)
            h_out = None
            for i in range(bpw):
                h_in.wait()
                if h_out is not None:
                    h_out.wait()
                shift()
                h_out = start_out(i)
                if i + 1 < bpw:
                    h_in = start_in(i + 1)
            h_out.wait()

    return sc_concat


def kernel(x, y, emb_table):
    B, C = x.shape[0], x.shape[1]
    R, D = emb_table.shape
    x3 = x.reshape(B, C, D)
    idx = y.reshape(-1).astype(jnp.int32)
    pad = jnp.broadcast_to(idx[:1], (_TILE - R,))
    idx8 = jnp.concatenate([idx, pad])
    return _build(B, C, D, R)(x3, idx8, emb_table)
